# Initial kernel scaffold; baseline (speedup 1.0000x reference)
import jax
import jax.numpy as jnp
from jax.experimental import pallas as pl


def kernel(edge_feature, emb0, emb1, emb2, emb3, W_amount, b_amount, W_time, b_time):
    raise NotImplementedError("write your pallas kernel here")


# speedup vs baseline: 2.1280x; 2.1280x over previous
import functools

import jax
import jax.numpy as jnp
from jax import lax
from jax.experimental import pallas as pl
from jax.experimental.pallas import tpu as pltpu
from jax.experimental.pallas import tpu_sc as plsc

E = 640000
NC, NS = 2, 16
NW = NC * NS
C = E // NW
BF = 128
NBATCH = C // BF + (1 if C % BF else 0)
INT_DIM = 32

_mesh = plsc.VectorSubcoreMesh(core_axis_name="c", subcore_axis_name="s")


def _body(i0, i1, i2, i3, amt, tm, t0, t1, t2, t3, linp, out,
          i0v, i1v, i2v, i3v, r0, r1, r2, r3, linb, av, tv, wv, sem):
    wid = lax.axis_index("s") * NC + lax.axis_index("c")
    base0 = wid * C
    pltpu.sync_copy(linp, wv)
    wa = [wv[pl.ds(h * 16, 16)] for h in range(4)]
    ba = [wv[pl.ds(64 + h * 16, 16)] for h in range(4)]
    wt = [wv[pl.ds(128 + h * 16, 16)] for h in range(4)]
    bt = [wv[pl.ds(192 + h * 16, 16)] for h in range(4)]

    def do_batch(k, _):
        base = base0 + jnp.minimum(k * BF, C - BF)
        pltpu.sync_copy(i0.at[pl.ds(base, BF)], i0v)
        pltpu.sync_copy(i1.at[pl.ds(base, BF)], i1v)
        pltpu.sync_copy(i2.at[pl.ds(base, BF)], i2v)
        pltpu.sync_copy(i3.at[pl.ds(base, BF)], i3v)
        pltpu.sync_copy(amt.at[pl.ds(base, BF)], av)
        pltpu.sync_copy(tm.at[pl.ds(base, BF)], tv)
        cps = [
            pltpu.async_copy(t0.at[i0v], r0, sem),
            pltpu.async_copy(t1.at[i1v], r1, sem),
            pltpu.async_copy(t2.at[i2v], r2, sem),
            pltpu.async_copy(t3.at[i3v], r3, sem),
        ]
        def lin_g(g, _):
            a16 = av[pl.ds(g * 16, 16)]
            t16 = tv[pl.ds(g * 16, 16)]
            for ee in range(16):
                a = jnp.full((16,), a16[ee], jnp.float32)
                t = jnp.full((16,), t16[ee], jnp.float32)
                e = g * 16 + ee
                for h in range(4):
                    linb[e, pl.ds(h * 16, 16)] = a * wa[h] + ba[h]
                    linb[e, pl.ds(64 + h * 16, 16)] = t * wt[h] + bt[h]
            return 0
        lax.fori_loop(0, BF // 16, lin_g, 0)
        for cp in cps:
            cp.wait()
        pltpu.sync_copy(r0, out.at[pl.ds(base, BF), pl.ds(0, 32)])
        pltpu.sync_copy(r1, out.at[pl.ds(base, BF), pl.ds(32, 32)])
        pltpu.sync_copy(r2, out.at[pl.ds(base, BF), pl.ds(64, 32)])
        pltpu.sync_copy(r3, out.at[pl.ds(base, BF), pl.ds(96, 32)])
        pltpu.sync_copy(linb, out.at[pl.ds(base, BF), pl.ds(128, 128)])
        return 0

    lax.fori_loop(0, NBATCH, do_batch, 0)


_sc_call = functools.partial(
    pl.kernel,
    out_type=jax.ShapeDtypeStruct((E, 256), jnp.float32),
    mesh=_mesh,
    compiler_params=pltpu.CompilerParams(use_tc_tiling_on_sc=False),
    scratch_types=[
        pltpu.VMEM((BF,), jnp.int32),
        pltpu.VMEM((BF,), jnp.int32),
        pltpu.VMEM((BF,), jnp.int32),
        pltpu.VMEM((BF,), jnp.int32),
        pltpu.VMEM((BF, INT_DIM), jnp.float32),
        pltpu.VMEM((BF, INT_DIM), jnp.float32),
        pltpu.VMEM((BF, INT_DIM), jnp.float32),
        pltpu.VMEM((BF, INT_DIM), jnp.float32),
        pltpu.VMEM((BF, 128), jnp.float32),
        pltpu.VMEM((BF,), jnp.float32),
        pltpu.VMEM((BF,), jnp.float32),
        pltpu.VMEM((256,), jnp.float32),
        pltpu.SemaphoreType.DMA,
    ],
)(_body)


def kernel(edge_feature, emb0, emb1, emb2, emb3, W_amount, b_amount, W_time, b_time):
    idx = edge_feature[:, :4].astype(jnp.int32)
    linp = jnp.concatenate([W_amount[0], b_amount, W_time[0], b_time])
    return _sc_call(idx[:, 0], idx[:, 1], idx[:, 2], idx[:, 3],
                    edge_feature[:, 4], edge_feature[:, 5],
                    emb0, emb1, emb2, emb3, linp)
